# Initial kernel scaffold; baseline (speedup 1.0000x reference)
import functools

import jax
import jax.numpy as jnp
from jax import lax
from jax.experimental import pallas as pl
from jax.experimental.pallas import tpu as pltpu
from jax.experimental.pallas import tpu_sc as plsc

VOCAB = 16
D_MODEL = 8


def _table_body(w_ref, e_ref, b_ref, tt_ref):
  tt = lax.dot_general(
      w_ref[...], e_ref[...],
      dimension_numbers=(((1,), (1,)), ((), ())),
      preferred_element_type=jnp.float32,
  )
  tt_ref[...] = tt + b_ref[...]


def _logit_table_t(W, embed, b):
  return pl.pallas_call(
      _table_body,
      out_shape=jax.ShapeDtypeStruct((VOCAB, VOCAB), jnp.float32),
  )(W, embed, b.reshape(VOCAB, 1))


def _make_sc_gather(n_tokens: int, chunk: int, n_workers: int, lanes: int):
  assert n_tokens % (n_workers * chunk) == 0
  per_worker = n_tokens // n_workers
  n_chunks = per_worker // chunk
  groups = chunk // lanes

  mesh = plsc.VectorSubcoreMesh(core_axis_name="c", subcore_axis_name="s")
  num_cores = mesh.num_cores

  @functools.partial(
      pl.kernel,
      out_type=jax.ShapeDtypeStruct((n_tokens * VOCAB,), jnp.float32),
      mesh=mesh,
      scratch_types=[
          pltpu.VMEM((VOCAB, VOCAB), jnp.float32),
          pltpu.VMEM((chunk,), jnp.int32),
          pltpu.VMEM((chunk * VOCAB,), jnp.float32),
      ],
  )
  def sc_gather(x_hbm, tt_hbm, out_hbm, tt_v, idx_v, out_v):
    wid = lax.axis_index("s") * num_cores + lax.axis_index("c")
    pltpu.sync_copy(tt_hbm, tt_v)

    lane_iota = lax.iota(jnp.int32, lanes)
    addr_base = lane_iota * VOCAB

    def group_body(g, carry):
      idxv = idx_v[pl.ds(g * lanes, lanes)]
      goff = g * (lanes * VOCAB)
      for u in range(VOCAB):
        u_vec = jnp.full((lanes,), u, jnp.int32)
        vals = plsc.load_gather(tt_v, [u_vec, idxv])
        plsc.store_scatter(out_v, [addr_base + (goff + u)], vals)
      return carry

    def chunk_body(c, carry):
      base = wid * per_worker + c * chunk
      pltpu.sync_copy(x_hbm.at[pl.ds(base, chunk)], idx_v)
      lax.fori_loop(0, groups, group_body, 0)
      pltpu.sync_copy(out_v, out_hbm.at[pl.ds(base * VOCAB, chunk * VOCAB)])
      return carry

    lax.fori_loop(0, n_chunks, chunk_body, 0)

  return sc_gather


def kernel(x, embed, W, b):
  B, T = x.shape
  n_tokens = B * T
  tt = _logit_table_t(W, embed, b)
  info = plsc.get_sparse_core_info()
  n_workers = info.num_cores * info.num_subcores
  sc_gather = _make_sc_gather(n_tokens, 2048, n_workers, info.num_lanes)
  out = sc_gather(x.reshape(n_tokens), tt)
  return out.reshape(B, T, VOCAB)


# speedup vs baseline: 6.0923x; 6.0923x over previous
import functools

import jax
import jax.numpy as jnp
from jax import lax
from jax.experimental import pallas as pl
from jax.experimental.pallas import tpu as pltpu
from jax.experimental.pallas import tpu_sc as plsc

VOCAB = 16
D_MODEL = 8


def _table_body(w_ref, e_ref, b_ref, tt_ref):
  tt = lax.dot_general(
      w_ref[...], e_ref[...],
      dimension_numbers=(((1,), (1,)), ((), ())),
      preferred_element_type=jnp.float32,
  )
  tt_ref[...] = tt + b_ref[...]


def _logit_table_t(W, embed, b):
  return pl.pallas_call(
      _table_body,
      out_shape=jax.ShapeDtypeStruct((VOCAB, VOCAB), jnp.float32),
  )(W, embed, b.reshape(VOCAB, 1))


def _make_sc_gather(n_tokens: int, chunk: int, n_workers: int, lanes: int):
  assert n_tokens % (n_workers * chunk) == 0
  per_worker = n_tokens // n_workers
  n_chunks = per_worker // chunk
  groups = chunk // lanes

  mesh = plsc.VectorSubcoreMesh(core_axis_name="c", subcore_axis_name="s")
  num_cores = mesh.num_cores

  @functools.partial(
      pl.kernel,
      out_type=jax.ShapeDtypeStruct((n_tokens * VOCAB,), jnp.float32),
      mesh=mesh,
      compiler_params=pltpu.CompilerParams(needs_layout_passes=False),
      scratch_types=[
          pltpu.VMEM((VOCAB * VOCAB,), jnp.float32),
          pltpu.VMEM((chunk,), jnp.int32),
          pltpu.VMEM((chunk * VOCAB,), jnp.float32),
      ],
  )
  def sc_gather(x_hbm, tt_hbm, out_hbm, tt_v, idx_v, out_v):
    wid = lax.axis_index("s") * num_cores + lax.axis_index("c")
    pltpu.sync_copy(tt_hbm, tt_v)

    lane_iota = lax.iota(jnp.int32, lanes)
    addr_base = lane_iota * VOCAB

    def group_body(g, carry):
      idxv = idx_v[pl.ds(g * lanes, lanes)]
      goff = g * (lanes * VOCAB)
      for u in range(VOCAB):
        vals = plsc.load_gather(tt_v, [idxv + (u * VOCAB)])
        plsc.store_scatter(out_v, [addr_base + (goff + u)], vals)
      return carry

    def chunk_body(c, carry):
      base = wid * per_worker + c * chunk
      pltpu.sync_copy(x_hbm.at[pl.ds(base, chunk)], idx_v)
      lax.fori_loop(0, groups, group_body, 0)
      pltpu.sync_copy(out_v, out_hbm.at[pl.ds(base * VOCAB, chunk * VOCAB)])
      return carry

    lax.fori_loop(0, n_chunks, chunk_body, 0)

  return sc_gather


def kernel(x, embed, W, b):
  B, T = x.shape
  n_tokens = B * T
  tt = _logit_table_t(W, embed, b)
  info = plsc.get_sparse_core_info()
  n_workers = info.num_cores * info.num_subcores
  sc_gather = _make_sc_gather(n_tokens, 2048, n_workers, info.num_lanes)
  out = sc_gather(x.reshape(n_tokens), tt.reshape(VOCAB * VOCAB))
  return out.reshape(B, T, VOCAB)


# speedup vs baseline: 6.8310x; 1.1212x over previous
import functools

import jax
import jax.numpy as jnp
from jax import lax
from jax.experimental import pallas as pl
from jax.experimental.pallas import tpu as pltpu
from jax.experimental.pallas import tpu_sc as plsc

VOCAB = 16
D_MODEL = 8


def _table_body(w_ref, e_ref, b_ref, tt_ref):
  tt = lax.dot_general(
      w_ref[...], e_ref[...],
      dimension_numbers=(((1,), (1,)), ((), ())),
      preferred_element_type=jnp.float32,
  )
  tt_ref[...] = tt + b_ref[...]


def _logit_table_t(W, embed, b):
  return pl.pallas_call(
      _table_body,
      out_shape=jax.ShapeDtypeStruct((VOCAB, VOCAB), jnp.float32),
  )(W, embed, b.reshape(VOCAB, 1))


def _make_sc_gather(n_tokens: int, chunk: int, n_workers: int, lanes: int):
  assert n_tokens % (n_workers * chunk) == 0
  per_worker = n_tokens // n_workers
  n_chunks = per_worker // chunk
  groups = chunk // lanes

  mesh = plsc.VectorSubcoreMesh(core_axis_name="c", subcore_axis_name="s")
  num_cores = mesh.num_cores

  @functools.partial(
      pl.kernel,
      out_type=jax.ShapeDtypeStruct((n_tokens * VOCAB,), jnp.float32),
      mesh=mesh,
      compiler_params=pltpu.CompilerParams(needs_layout_passes=False),
      scratch_types=[
          pltpu.VMEM((VOCAB * VOCAB,), jnp.float32),
          pltpu.VMEM((chunk,), jnp.int32),
          pltpu.VMEM((chunk,), jnp.int32),
          pltpu.VMEM((chunk * VOCAB,), jnp.float32),
          pltpu.VMEM((chunk * VOCAB,), jnp.float32),
          pltpu.SemaphoreType.DMA,
          pltpu.SemaphoreType.DMA,
          pltpu.SemaphoreType.DMA,
          pltpu.SemaphoreType.DMA,
      ],
  )
  def sc_gather(x_hbm, tt_hbm, out_hbm, tt_v, idx0, idx1, out0, out1,
                isem0, isem1, osem0, osem1):
    wid = lax.axis_index("s") * num_cores + lax.axis_index("c")
    pltpu.sync_copy(tt_hbm, tt_v)

    idx_bufs, out_bufs = [idx0, idx1], [out0, out1]
    isems, osems = [isem0, isem1], [osem0, osem1]

    lane_iota = lax.iota(jnp.int32, lanes)
    addr_base = lane_iota * VOCAB

    def idx_copy(c, buf):
      base = wid * per_worker + c * chunk
      return pltpu.make_async_copy(
          x_hbm.at[pl.ds(base, chunk)], idx_bufs[buf], isems[buf])

    def out_copy(c, buf):
      base = wid * per_worker + c * chunk
      return pltpu.make_async_copy(
          out_bufs[buf], out_hbm.at[pl.ds(base * VOCAB, chunk * VOCAB)],
          osems[buf])

    def compute(ibuf, obuf):
      @plsc.parallel_loop(0, groups, unroll=2)
      def _(g):
        idxv = ibuf[pl.ds(g * lanes, lanes)]
        goff = g * (lanes * VOCAB)
        for u in range(VOCAB):
          vals = plsc.load_gather(tt_v, [idxv + (u * VOCAB)])
          plsc.store_scatter(obuf, [addr_base + (goff + u)], vals)

    idx_copy(0, 0).start()
    for c in range(n_chunks):
      buf = c % 2
      if c + 1 < n_chunks:
        idx_copy(c + 1, 1 - buf).start()
      idx_copy(c, buf).wait()
      if c >= 2:
        out_copy(c - 2, buf).wait()
      compute(idx_bufs[buf], out_bufs[buf])
      out_copy(c, buf).start()
    out_copy(n_chunks - 2, n_chunks % 2).wait()
    out_copy(n_chunks - 1, 1 - n_chunks % 2).wait()

  return sc_gather


def kernel(x, embed, W, b):
  B, T = x.shape
  n_tokens = B * T
  tt = _logit_table_t(W, embed, b)
  info = plsc.get_sparse_core_info()
  n_workers = info.num_cores * info.num_subcores
  sc_gather = _make_sc_gather(n_tokens, 2048, n_workers, info.num_lanes)
  out = sc_gather(x.reshape(n_tokens), tt.reshape(VOCAB * VOCAB))
  return out.reshape(B, T, VOCAB)


# speedup vs baseline: 7.1827x; 1.0515x over previous
import functools

import jax
import jax.numpy as jnp
from jax import lax
from jax.experimental import pallas as pl
from jax.experimental.pallas import tpu as pltpu
from jax.experimental.pallas import tpu_sc as plsc

VOCAB = 16
D_MODEL = 8


def _table_body(w_ref, e_ref, b_ref, tt_ref):
  tt = lax.dot_general(
      w_ref[...], e_ref[...],
      dimension_numbers=(((1,), (1,)), ((), ())),
      preferred_element_type=jnp.float32,
  )
  tt_ref[...] = tt + b_ref[...]


def _logit_table_t(W, embed, b):
  tt = pl.pallas_call(
      _table_body,
      out_shape=jax.ShapeDtypeStruct((VOCAB, VOCAB), jnp.float32),
  )(W, embed, b.reshape(VOCAB, 1))
  return tt.reshape(VOCAB * VOCAB)


def _make_sc_gather(B: int, T: int, chunk: int, n_workers: int, lanes: int):
  n_tokens = B * T
  assert n_tokens % (n_workers * chunk) == 0 and T % chunk == 0
  per_worker = n_tokens // n_workers
  rows_per_worker = per_worker // T
  assert rows_per_worker * T == per_worker
  chunks_per_row = T // chunk
  groups = chunk // lanes

  mesh = plsc.VectorSubcoreMesh(core_axis_name="c", subcore_axis_name="s")
  num_cores = mesh.num_cores

  @functools.partial(
      pl.kernel,
      out_type=jax.ShapeDtypeStruct((B, T, VOCAB), jnp.float32),
      mesh=mesh,
      compiler_params=pltpu.CompilerParams(
          needs_layout_passes=False, use_tc_tiling_on_sc=False),
      scratch_types=[
          pltpu.VMEM((VOCAB * VOCAB,), jnp.float32),
          pltpu.VMEM((chunk,), jnp.int32),
          pltpu.VMEM((chunk,), jnp.int32),
          pltpu.VMEM((chunk, VOCAB), jnp.float32),
          pltpu.VMEM((chunk, VOCAB), jnp.float32),
          pltpu.SemaphoreType.DMA,
          pltpu.SemaphoreType.DMA,
          pltpu.SemaphoreType.DMA,
          pltpu.SemaphoreType.DMA,
      ],
  )
  def sc_gather(x_hbm, tt_hbm, out_hbm, tt_v, idx0, idx1, out0, out1,
                isem0, isem1, osem0, osem1):
    wid = lax.axis_index("s") * num_cores + lax.axis_index("c")
    row0 = wid * rows_per_worker
    pltpu.sync_copy(tt_hbm, tt_v)

    idx_bufs, out_bufs = [idx0, idx1], [out0, out1]
    isems, osems = [isem0, isem1], [osem0, osem1]

    lane_iota = lax.iota(jnp.int32, lanes)

    def idx_copy(c, buf):
      row = row0 + c // chunks_per_row
      col = (c % chunks_per_row) * chunk
      return pltpu.make_async_copy(
          x_hbm.at[row, pl.ds(col, chunk)], idx_bufs[buf], isems[buf])

    def out_copy(c, buf):
      row = row0 + c // chunks_per_row
      col = (c % chunks_per_row) * chunk
      return pltpu.make_async_copy(
          out_bufs[buf], out_hbm.at[row, pl.ds(col, chunk)], osems[buf])

    def compute(ibuf, obuf):
      @plsc.parallel_loop(0, groups, unroll=2)
      def _(g):
        idxv = ibuf[pl.ds(g * lanes, lanes)]
        rows = lane_iota + g * lanes
        for u in range(VOCAB):
          vals = plsc.load_gather(tt_v, [idxv + (u * VOCAB)])
          plsc.store_scatter(obuf, [rows, jnp.full((lanes,), u, jnp.int32)],
                             vals)

    n_chunks = rows_per_worker * chunks_per_row
    idx_copy(0, 0).start()
    for c in range(n_chunks):
      buf = c % 2
      if c + 1 < n_chunks:
        idx_copy(c + 1, 1 - buf).start()
      idx_copy(c, buf).wait()
      if c >= 2:
        out_copy(c - 2, buf).wait()
      compute(idx_bufs[buf], out_bufs[buf])
      out_copy(c, buf).start()
    out_copy(n_chunks - 2, n_chunks % 2).wait()
    out_copy(n_chunks - 1, 1 - n_chunks % 2).wait()

  return sc_gather


def kernel(x, embed, W, b):
  B, T = x.shape
  tt = _logit_table_t(W, embed, b)
  info = plsc.get_sparse_core_info()
  n_workers = info.num_cores * info.num_subcores
  sc_gather = _make_sc_gather(B, T, 2048, n_workers, info.num_lanes)
  return sc_gather(x, tt)


# speedup vs baseline: 7.4157x; 1.0324x over previous
import functools

import jax
import jax.numpy as jnp
from jax import lax
from jax.experimental import pallas as pl
from jax.experimental.pallas import tpu as pltpu
from jax.experimental.pallas import tpu_sc as plsc

VOCAB = 16
D_MODEL = 8


def _table_body(w_ref, e_ref, b_ref, tt_ref):
  tt = lax.dot_general(
      w_ref[...], e_ref[...],
      dimension_numbers=(((1,), (1,)), ((), ())),
      preferred_element_type=jnp.float32,
  )
  tt_ref[...] = tt + b_ref[...]


def _logit_table_t(W, embed, b):
  tt = pl.pallas_call(
      _table_body,
      out_shape=jax.ShapeDtypeStruct((VOCAB, VOCAB), jnp.float32),
  )(W, embed, b.reshape(VOCAB, 1))
  return tt.reshape(VOCAB * VOCAB)


def _make_sc_gather(B: int, T: int, chunk: int, n_workers: int, lanes: int):
  n_tokens = B * T
  assert n_tokens % (n_workers * chunk) == 0 and T % chunk == 0
  per_worker = n_tokens // n_workers
  rows_per_worker = per_worker // T
  assert rows_per_worker * T == per_worker
  chunks_per_row = T // chunk
  groups = chunk // lanes

  mesh = plsc.VectorSubcoreMesh(core_axis_name="c", subcore_axis_name="s")
  num_cores = mesh.num_cores

  vals_per_wide_row = 128
  wide_rows = n_tokens * VOCAB // vals_per_wide_row
  chunk_wide_rows = chunk * VOCAB // vals_per_wide_row

  @functools.partial(
      pl.kernel,
      out_type=jax.ShapeDtypeStruct((wide_rows, vals_per_wide_row),
                                    jnp.float32),
      mesh=mesh,
      compiler_params=pltpu.CompilerParams(
          needs_layout_passes=False, use_tc_tiling_on_sc=False),
      scratch_types=[
          pltpu.VMEM((VOCAB * VOCAB,), jnp.float32),
          pltpu.VMEM((chunk,), jnp.int32),
          pltpu.VMEM((chunk,), jnp.int32),
          pltpu.VMEM((chunk_wide_rows, vals_per_wide_row), jnp.float32),
          pltpu.VMEM((chunk_wide_rows, vals_per_wide_row), jnp.float32),
          pltpu.SemaphoreType.DMA,
          pltpu.SemaphoreType.DMA,
          pltpu.SemaphoreType.DMA,
          pltpu.SemaphoreType.DMA,
      ],
  )
  def sc_gather(x_hbm, tt_hbm, out_hbm, tt_v, idx0, idx1, out0, out1,
                isem0, isem1, osem0, osem1):
    wid = lax.axis_index("s") * num_cores + lax.axis_index("c")
    row0 = wid * rows_per_worker
    pltpu.sync_copy(tt_hbm, tt_v)

    idx_bufs, out_bufs = [idx0, idx1], [out0, out1]
    isems, osems = [isem0, isem1], [osem0, osem1]

    lane_iota = lax.iota(jnp.int32, lanes)

    def idx_copy(c, buf):
      row = row0 + c // chunks_per_row
      col = (c % chunks_per_row) * chunk
      return pltpu.make_async_copy(
          x_hbm.at[row, pl.ds(col, chunk)], idx_bufs[buf], isems[buf])

    def out_copy(c, buf):
      row = row0 + c // chunks_per_row
      col = (c % chunks_per_row) * chunk
      wrow = (row * T + col) * VOCAB // vals_per_wide_row
      return pltpu.make_async_copy(
          out_bufs[buf], out_hbm.at[pl.ds(wrow, chunk_wide_rows)],
          osems[buf])

    row_in_group = lane_iota // (vals_per_wide_row // VOCAB)
    col_of_lane = (lane_iota % (vals_per_wide_row // VOCAB)) * VOCAB
    wide_rows_per_group = lanes * VOCAB // vals_per_wide_row

    def compute(ibuf, obuf):
      @plsc.parallel_loop(0, groups, unroll=2)
      def _(g):
        idxv = ibuf[pl.ds(g * lanes, lanes)]
        rows = row_in_group + g * wide_rows_per_group
        for u in range(VOCAB):
          vals = plsc.load_gather(tt_v, [idxv + (u * VOCAB)])
          plsc.store_scatter(obuf, [rows, col_of_lane + u], vals)

    n_chunks = rows_per_worker * chunks_per_row
    idx_copy(0, 0).start()
    for c in range(n_chunks):
      buf = c % 2
      if c + 1 < n_chunks:
        idx_copy(c + 1, 1 - buf).start()
      idx_copy(c, buf).wait()
      if c >= 2:
        out_copy(c - 2, buf).wait()
      compute(idx_bufs[buf], out_bufs[buf])
      out_copy(c, buf).start()
    out_copy(n_chunks - 2, n_chunks % 2).wait()
    out_copy(n_chunks - 1, 1 - n_chunks % 2).wait()

  return sc_gather


def kernel(x, embed, W, b):
  B, T = x.shape
  tt = _logit_table_t(W, embed, b)
  info = plsc.get_sparse_core_info()
  n_workers = info.num_cores * info.num_subcores
  sc_gather = _make_sc_gather(B, T, 2048, n_workers, info.num_lanes)
  out = sc_gather(x, tt)
  return out.reshape(B, T, VOCAB)


# speedup vs baseline: 59.6152x; 8.0390x over previous
import functools

import jax
import jax.numpy as jnp
from jax import lax
from jax.experimental import pallas as pl
from jax.experimental.pallas import tpu as pltpu
from jax.experimental.pallas import tpu_sc as plsc

VOCAB = 16
D_MODEL = 8
LANES = 128
SUBL = 8


def _table_body(w_ref, e_ref, b_ref, tt_ref):
  tt = lax.dot_general(
      w_ref[...], e_ref[...],
      dimension_numbers=(((1,), (1,)), ((), ())),
      preferred_element_type=jnp.float32,
  )
  tt_ref[...] = tt + b_ref[...]


def _logit_table_t(W, embed, b):
  tt = pl.pallas_call(
      _table_body,
      out_shape=jax.ShapeDtypeStruct((VOCAB, VOCAB), jnp.float32),
  )(W, embed, b.reshape(VOCAB, 1))
  return tt.reshape(VOCAB * VOCAB)


def _make_sc_gather(B: int, T: int, chunk: int, n_workers: int, lanes: int):
  n_tokens = B * T
  assert n_tokens % (n_workers * chunk) == 0 and T % chunk == 0
  assert chunk % LANES == 0
  per_worker = n_tokens // n_workers
  rows_per_worker = per_worker // T
  assert rows_per_worker * T == per_worker
  chunks_per_row = T // chunk
  groups = chunk // lanes
  half_words = chunk * SUBL
  b_words = T * VOCAB
  v8_words = T * SUBL

  mesh = plsc.VectorSubcoreMesh(core_axis_name="c", subcore_axis_name="s")
  num_cores = mesh.num_cores

  @functools.partial(
      pl.kernel,
      out_type=jax.ShapeDtypeStruct((n_tokens * VOCAB,), jnp.float32),
      mesh=mesh,
      compiler_params=pltpu.CompilerParams(
          needs_layout_passes=False, use_tc_tiling_on_sc=False),
      scratch_types=[
          pltpu.VMEM((VOCAB * VOCAB,), jnp.float32),
          pltpu.VMEM((chunk,), jnp.int32),
          pltpu.VMEM((chunk,), jnp.int32),
          pltpu.VMEM((chunk * VOCAB,), jnp.float32),
          pltpu.VMEM((chunk * VOCAB,), jnp.float32),
          pltpu.SemaphoreType.DMA,
          pltpu.SemaphoreType.DMA,
          pltpu.SemaphoreType.DMA,
          pltpu.SemaphoreType.DMA,
      ],
  )
  def sc_gather(x_hbm, tt_hbm, out_hbm, tt_v, idx0, idx1, out0, out1,
                isem0, isem1, osem0, osem1):
    wid = lax.axis_index("s") * num_cores + lax.axis_index("c")
    row0 = wid * rows_per_worker
    pltpu.sync_copy(tt_hbm, tt_v)

    idx_bufs, out_bufs = [idx0, idx1], [out0, out1]
    isems, osems = [isem0, isem1], [osem0, osem1]

    def idx_copy(c, buf):
      row = row0 + c // chunks_per_row
      col = (c % chunks_per_row) * chunk
      return pltpu.make_async_copy(
          x_hbm.at[row, pl.ds(col, chunk)], idx_bufs[buf], isems[buf])

    def out_copy(c, buf, v8):
      row = row0 + c // chunks_per_row
      col = (c % chunks_per_row) * chunk
      off = row * b_words + v8 * v8_words + col * SUBL
      return pltpu.make_async_copy(
          out_bufs[buf].at[pl.ds(v8 * half_words, half_words)],
          out_hbm.at[pl.ds(off, half_words)], osems[buf])

    def compute(ibuf, obuf):
      @plsc.parallel_loop(0, groups, unroll=2)
      def _(g):
        idxv = ibuf[pl.ds(g * lanes, lanes)]
        gbase = (g // SUBL) * (SUBL * LANES) + (g % SUBL) * lanes
        for v in range(VOCAB):
          vals = plsc.load_gather(tt_v, [idxv + (v * VOCAB)])
          voff = (v // SUBL) * half_words + (v % SUBL) * LANES
          obuf[pl.ds(gbase + voff, lanes)] = vals

    n_chunks = rows_per_worker * chunks_per_row
    idx_copy(0, 0).start()
    for c in range(n_chunks):
      buf = c % 2
      if c + 1 < n_chunks:
        idx_copy(c + 1, 1 - buf).start()
      idx_copy(c, buf).wait()
      if c >= 2:
        out_copy(c - 2, buf, 0).wait()
        out_copy(c - 2, buf, 1).wait()
      compute(idx_bufs[buf], out_bufs[buf])
      out_copy(c, buf, 0).start()
      out_copy(c, buf, 1).start()
    for c in (n_chunks - 2, n_chunks - 1):
      out_copy(c, c % 2, 0).wait()
      out_copy(c, c % 2, 1).wait()

  return sc_gather


def kernel(x, embed, W, b):
  B, T = x.shape
  tt = _logit_table_t(W, embed, b)
  info = plsc.get_sparse_core_info()
  n_workers = info.num_cores * info.num_subcores
  sc_gather = _make_sc_gather(B, T, 2048, n_workers, info.num_lanes)
  out = sc_gather(x, tt)
  out = out.reshape(B, VOCAB // SUBL, T // LANES, SUBL, LANES)
  return out.transpose(0, 2, 4, 1, 3).reshape(B, T, VOCAB)


# speedup vs baseline: 64.5082x; 1.0821x over previous
import functools

import jax
import jax.numpy as jnp
from jax import lax
from jax.experimental import pallas as pl
from jax.experimental.pallas import tpu as pltpu
from jax.experimental.pallas import tpu_sc as plsc

VOCAB = 16
D_MODEL = 8
LANES = 128
SUBL = 8


def _table_body(w_ref, e_ref, b_ref, tt_ref):
  tt = lax.dot_general(
      w_ref[...], e_ref[...],
      dimension_numbers=(((1,), (1,)), ((), ())),
      preferred_element_type=jnp.float32,
  )
  tt_ref[...] = tt + b_ref[...]


def _logit_table_t(W, embed, b):
  tt = pl.pallas_call(
      _table_body,
      out_shape=jax.ShapeDtypeStruct((VOCAB, VOCAB), jnp.float32),
  )(W, embed, b.reshape(VOCAB, 1))
  return tt.reshape(VOCAB * VOCAB)


def _make_sc_gather(B: int, T: int, chunk: int, n_workers: int, lanes: int):
  n_tokens = B * T
  assert n_tokens % (n_workers * chunk) == 0 and T % chunk == 0
  assert chunk % LANES == 0
  per_worker = n_tokens // n_workers
  rows_per_worker = per_worker // T
  assert rows_per_worker * T == per_worker
  chunks_per_row = T // chunk
  groups = chunk // lanes
  half_words = chunk * SUBL
  b_words = T * VOCAB
  v8_words = T * SUBL

  mesh = plsc.VectorSubcoreMesh(core_axis_name="c", subcore_axis_name="s")
  num_cores = mesh.num_cores

  @functools.partial(
      pl.kernel,
      out_type=jax.ShapeDtypeStruct((n_tokens * VOCAB,), jnp.float32),
      mesh=mesh,
      compiler_params=pltpu.CompilerParams(needs_layout_passes=False),
      scratch_types=[
          pltpu.VMEM((VOCAB * VOCAB,), jnp.float32),
          pltpu.VMEM((chunk,), jnp.int32),
          pltpu.VMEM((chunk,), jnp.int32),
          pltpu.VMEM((chunk * VOCAB,), jnp.float32),
          pltpu.VMEM((chunk * VOCAB,), jnp.float32),
          pltpu.SemaphoreType.DMA,
          pltpu.SemaphoreType.DMA,
          pltpu.SemaphoreType.DMA,
          pltpu.SemaphoreType.DMA,
      ],
  )
  def sc_gather(x_hbm, tt_hbm, out_hbm, tt_v, idx0, idx1, out0, out1,
                isem0, isem1, osem0, osem1):
    wid = lax.axis_index("s") * num_cores + lax.axis_index("c")
    row0 = wid * rows_per_worker
    pltpu.sync_copy(tt_hbm, tt_v)

    idx_bufs, out_bufs = [idx0, idx1], [out0, out1]
    isems, osems = [isem0, isem1], [osem0, osem1]

    def idx_copy(c, buf):
      row = row0 + c // chunks_per_row
      col = (c % chunks_per_row) * chunk
      return pltpu.make_async_copy(
          x_hbm.at[row, pl.ds(col, chunk)], idx_bufs[buf], isems[buf])

    def out_copy(c, buf, v8):
      row = row0 + c // chunks_per_row
      col = (c % chunks_per_row) * chunk
      off = row * b_words + v8 * v8_words + col * SUBL
      return pltpu.make_async_copy(
          out_bufs[buf].at[pl.ds(v8 * half_words, half_words)],
          out_hbm.at[pl.ds(off, half_words)], osems[buf])

    def compute(ibuf, obuf):
      @plsc.parallel_loop(0, groups, unroll=2)
      def _(g):
        idxv = ibuf[pl.ds(g * lanes, lanes)]
        gbase = (g // SUBL) * (SUBL * LANES) + (g % SUBL) * lanes
        for v in range(VOCAB):
          vals = plsc.load_gather(tt_v, [idxv + (v * VOCAB)])
          voff = (v // SUBL) * half_words + (v % SUBL) * LANES
          obuf[pl.ds(gbase + voff, lanes)] = vals

    n_chunks = rows_per_worker * chunks_per_row
    idx_copy(0, 0).start()
    for c in range(n_chunks):
      buf = c % 2
      if c + 1 < n_chunks:
        idx_copy(c + 1, 1 - buf).start()
      idx_copy(c, buf).wait()
      if c >= 2:
        out_copy(c - 2, buf, 0).wait()
        out_copy(c - 2, buf, 1).wait()
      compute(idx_bufs[buf], out_bufs[buf])
      out_copy(c, buf, 0).start()
      out_copy(c, buf, 1).start()
    for c in (n_chunks - 2, n_chunks - 1):
      out_copy(c, c % 2, 0).wait()
      out_copy(c, c % 2, 1).wait()

  return sc_gather


def kernel(x, embed, W, b):
  B, T = x.shape
  tt = _logit_table_t(W, embed, b)
  info = plsc.get_sparse_core_info()
  n_workers = info.num_cores * info.num_subcores
  sc_gather = _make_sc_gather(B, T, 2048, n_workers, info.num_lanes)
  out = sc_gather(x, tt)
  out = out.reshape(B, VOCAB // SUBL, T // LANES, SUBL, LANES)
  return out.transpose(0, 2, 4, 1, 3).reshape(B, T, VOCAB)


# speedup vs baseline: 64.6949x; 1.0029x over previous
import functools

import jax
import jax.numpy as jnp
from jax import lax
from jax.experimental import pallas as pl
from jax.experimental.pallas import tpu as pltpu
from jax.experimental.pallas import tpu_sc as plsc

VOCAB = 16
D_MODEL = 8
LANES = 128
SUBL = 8


def _make_sc_kernel(B: int, T: int, chunk: int, n_workers: int, lanes: int):
  n_tokens = B * T
  assert n_tokens % (n_workers * chunk) == 0 and T % chunk == 0
  assert chunk % LANES == 0
  per_worker = n_tokens // n_workers
  rows_per_worker = per_worker // T
  assert rows_per_worker * T == per_worker
  chunks_per_row = T // chunk
  groups = chunk // lanes
  half_words = chunk * SUBL
  b_words = T * VOCAB
  v8_words = T * SUBL

  mesh = plsc.VectorSubcoreMesh(core_axis_name="c", subcore_axis_name="s")
  num_cores = mesh.num_cores

  @functools.partial(
      pl.kernel,
      out_type=jax.ShapeDtypeStruct((n_tokens * VOCAB,), jnp.float32),
      mesh=mesh,
      compiler_params=pltpu.CompilerParams(needs_layout_passes=False),
      scratch_types=[
          pltpu.VMEM((2 * VOCAB * D_MODEL + VOCAB,), jnp.float32),
          pltpu.VMEM((VOCAB * VOCAB,), jnp.float32),
          pltpu.VMEM((chunk,), jnp.int32),
          pltpu.VMEM((chunk,), jnp.int32),
          pltpu.VMEM((chunk * VOCAB,), jnp.float32),
          pltpu.VMEM((chunk * VOCAB,), jnp.float32),
          pltpu.SemaphoreType.DMA,
          pltpu.SemaphoreType.DMA,
          pltpu.SemaphoreType.DMA,
          pltpu.SemaphoreType.DMA,
          pltpu.SemaphoreType.DMA,
      ],
  )
  def sc_kernel(x_hbm, e_hbm, w_hbm, b_hbm, out_hbm, ew_v, tt_v,
                idx0, idx1, out0, out1, isem0, isem1, osem0, osem1, wsem):
    wid = lax.axis_index("s") * num_cores + lax.axis_index("c")
    row0 = wid * rows_per_worker

    idx_bufs, out_bufs = [idx0, idx1], [out0, out1]
    isems, osems = [isem0, isem1], [osem0, osem1]

    def idx_copy(c, buf):
      row = row0 + c // chunks_per_row
      col = (c % chunks_per_row) * chunk
      return pltpu.make_async_copy(
          x_hbm.at[row, pl.ds(col, chunk)], idx_bufs[buf], isems[buf])

    idx_copy(0, 0).start()

    stage = [
        pltpu.make_async_copy(
            e_hbm, ew_v.at[pl.ds(0, VOCAB * D_MODEL)], wsem),
        pltpu.make_async_copy(
            w_hbm, ew_v.at[pl.ds(VOCAB * D_MODEL, VOCAB * D_MODEL)], wsem),
        pltpu.make_async_copy(
            b_hbm, ew_v.at[pl.ds(2 * VOCAB * D_MODEL, VOCAB)], wsem),
    ]
    for d in stage:
      d.start()
    for d in stage:
      d.wait()

    lane16 = lax.iota(jnp.int32, lanes)
    e_cols = [plsc.load_gather(ew_v, [lane16 * D_MODEL + d])
              for d in range(D_MODEL)]
    for u in range(VOCAB):
      acc = plsc.load_gather(
          ew_v, [jnp.full((lanes,), 2 * VOCAB * D_MODEL + u, jnp.int32)])
      for d in range(D_MODEL):
        wbc = plsc.load_gather(
            ew_v,
            [jnp.full((lanes,), VOCAB * D_MODEL + u * D_MODEL + d,
                      jnp.int32)])
        acc = acc + e_cols[d] * wbc
      tt_v[pl.ds(u * VOCAB, VOCAB)] = acc

    def out_copy(c, buf, v8):
      row = row0 + c // chunks_per_row
      col = (c % chunks_per_row) * chunk
      off = row * b_words + v8 * v8_words + col * SUBL
      return pltpu.make_async_copy(
          out_bufs[buf].at[pl.ds(v8 * half_words, half_words)],
          out_hbm.at[pl.ds(off, half_words)], osems[buf])

    def compute(ibuf, obuf):
      @plsc.parallel_loop(0, groups, unroll=2)
      def _(g):
        idxv = ibuf[pl.ds(g * lanes, lanes)]
        gbase = (g // SUBL) * (SUBL * LANES) + (g % SUBL) * lanes
        for v in range(VOCAB):
          vals = plsc.load_gather(tt_v, [idxv + (v * VOCAB)])
          voff = (v // SUBL) * half_words + (v % SUBL) * LANES
          obuf[pl.ds(gbase + voff, lanes)] = vals

    n_chunks = rows_per_worker * chunks_per_row
    for c in range(n_chunks):
      buf = c % 2
      if c + 1 < n_chunks:
        idx_copy(c + 1, 1 - buf).start()
      idx_copy(c, buf).wait()
      if c >= 2:
        out_copy(c - 2, buf, 0).wait()
        out_copy(c - 2, buf, 1).wait()
      compute(idx_bufs[buf], out_bufs[buf])
      out_copy(c, buf, 0).start()
      out_copy(c, buf, 1).start()
    for c in (n_chunks - 2, n_chunks - 1):
      out_copy(c, c % 2, 0).wait()
      out_copy(c, c % 2, 1).wait()

  return sc_kernel


def kernel(x, embed, W, b):
  B, T = x.shape
  info = plsc.get_sparse_core_info()
  n_workers = info.num_cores * info.num_subcores
  sc_kernel = _make_sc_kernel(B, T, 2048, n_workers, info.num_lanes)
  out = sc_kernel(x, embed.reshape(VOCAB * D_MODEL), W.reshape(VOCAB * D_MODEL),
                  b)
  out = out.reshape(B, VOCAB // SUBL, T // LANES, SUBL, LANES)
  return out.transpose(0, 2, 4, 1, 3).reshape(B, T, VOCAB)
